# baseline (speedup 1.0000x reference)
import jax
import jax.numpy as jnp
from jax.experimental import pallas as pl

_BS, _NV, _NP, _FD = 64, 4, 512, 10
_THR = 6.0
_SCALE = 1.0


def _chamfer_body(goal_ref, srows_ref, gpk_ref, out_ref):
    acc = None
    for v in range(_NV):
        g = goal_ref[0, v]
        sA = srows_ref[0, v]

        P = None
        for f in range(4):
            d = g[:, 5 + f:6 + f] - sA[2 + f:3 + f, :]
            P = d * d if P is None else P + d * d

        spk = jax.lax.bitcast_convert_type(sA[6:7, :], jnp.int32)
        gpk = jax.lax.bitcast_convert_type(gpk_ref[0, v], jnp.int32)

        minv_g = jnp.min(P, axis=1, keepdims=True)
        sel = P == minv_g
        q1 = jnp.sum(jnp.where(sel, spk, 0), axis=1, keepdims=True)
        q1r = jnp.reshape(q1, (1, _NP))
        m1r = jnp.reshape(minv_g, (1, _NP))
        sx = jax.lax.bitcast_convert_type(q1r & -65536, jnp.float32)
        sy = jax.lax.bitcast_convert_type(q1r << 16, jnp.float32)
        dx = sA[7:8, :] - sx
        dy = sA[8:9, :] - sy
        xy1 = jnp.where(m1r > _THR, 1.0, jnp.sqrt(dx * dx + dy * dy))

        minv_s = jnp.min(P, axis=0, keepdims=True)
        sel2 = P == minv_s
        q2 = jnp.sum(jnp.where(sel2, gpk, 0), axis=0, keepdims=True)
        gx = jax.lax.bitcast_convert_type(q2 & -65536, jnp.float32)
        gy = jax.lax.bitcast_convert_type(q2 << 16, jnp.float32)
        dx2 = sA[0:1, :] - gx
        dy2 = sA[1:2, :] - gy
        xy2 = jnp.where(minv_s > _THR, 1.0, jnp.sqrt(dx2 * dx2 + dy2 * dy2))

        part = xy1 + xy2
        acc = part if acc is None else acc + part

    total = jnp.sum(acc)
    out_ref[...] = (total * (-_SCALE / (2.0 * _NP * _NV))).reshape(1, 1, 1)


def _pack_xy(t):
    xb = t[..., 0].astype(jnp.bfloat16)
    yb = t[..., 1].astype(jnp.bfloat16)
    xu = jax.lax.bitcast_convert_type(xb, jnp.uint16).astype(jnp.uint32)
    yu = jax.lax.bitcast_convert_type(yb, jnp.uint16).astype(jnp.uint32)
    return jax.lax.bitcast_convert_type((xu << 16) | yu, jnp.float32)


def _row(t, i):
    return t[..., i][..., None, :]


@jax.jit
def kernel(achieved_goal, desired_goal):
    srows = jnp.concatenate(
        [_row(achieved_goal, 0), _row(achieved_goal, 1),
         _row(achieved_goal, 5), _row(achieved_goal, 6),
         _row(achieved_goal, 7), _row(achieved_goal, 8),
         _pack_xy(achieved_goal)[..., None, :],
         _row(desired_goal, 0), _row(desired_goal, 1)],
        axis=-2)
    gpk = _pack_xy(desired_goal)[..., None]
    out = pl.pallas_call(
        _chamfer_body,
        grid=(_BS,),
        in_specs=[
            pl.BlockSpec((1, _NV, _NP, _FD), lambda b: (b, 0, 0, 0)),
            pl.BlockSpec((1, _NV, 9, _NP), lambda b: (b, 0, 0, 0)),
            pl.BlockSpec((1, _NV, _NP, 1), lambda b: (b, 0, 0, 0)),
        ],
        out_specs=pl.BlockSpec((1, 1, 1), lambda b: (b, 0, 0)),
        out_shape=jax.ShapeDtypeStruct((_BS, 1, 1), jnp.float32),
    )(desired_goal, srows, gpk)
    return out.reshape(_BS, 1)


# speedup vs baseline: 1.4176x; 1.4176x over previous
import jax
import jax.numpy as jnp
from jax.experimental import pallas as pl

_BS, _NV, _NP, _FD = 64, 4, 512, 10
_THR = 6.0
_SCALE = 1.0


def _chamfer_body(goal_ref, state_ref, out_ref):
    acc = None
    for v in range(_NV):
        g = goal_ref[0, v]
        sT = jnp.swapaxes(state_ref[0, v], 0, 1)

        P = None
        for f in range(5, 9):
            d = g[:, f:f + 1] - sT[f:f + 1, :]
            P = d * d if P is None else P + d * d

        ex = g[:, 0:1] - sT[0:1, :]
        ey = g[:, 1:2] - sT[1:2, :]
        D2 = ex * ex + ey * ey

        minv_g = jnp.min(P, axis=1, keepdims=True)
        sel = P == minv_g
        q1 = jnp.sum(jnp.where(sel, D2, 0.0), axis=1, keepdims=True)
        q1r = jnp.reshape(q1, (1, _NP))
        m1r = jnp.reshape(minv_g, (1, _NP))
        xy1 = jnp.where(m1r > _THR, 1.0, jnp.sqrt(q1r))

        minv_s = jnp.min(P, axis=0, keepdims=True)
        sel2 = P == minv_s
        q2 = jnp.sum(jnp.where(sel2, D2, 0.0), axis=0, keepdims=True)
        xy2 = jnp.where(minv_s > _THR, 1.0, jnp.sqrt(q2))

        part = xy1 + xy2
        acc = part if acc is None else acc + part

    total = jnp.sum(acc)
    out_ref[...] = (total * (-_SCALE / (2.0 * _NP * _NV))).reshape(1, 1, 1)


@jax.jit
def kernel(achieved_goal, desired_goal):
    out = pl.pallas_call(
        _chamfer_body,
        grid=(_BS,),
        in_specs=[
            pl.BlockSpec((1, _NV, _NP, _FD), lambda b: (b, 0, 0, 0)),
            pl.BlockSpec((1, _NV, _NP, _FD), lambda b: (b, 0, 0, 0)),
        ],
        out_specs=pl.BlockSpec((1, 1, 1), lambda b: (b, 0, 0)),
        out_shape=jax.ShapeDtypeStruct((_BS, 1, 1), jnp.float32),
    )(desired_goal, achieved_goal)
    return out.reshape(_BS, 1)
